# Initial kernel scaffold; baseline (speedup 1.0000x reference)
import functools

import jax
import jax.numpy as jnp
from jax import lax
from jax.experimental import pallas as pl
from jax.experimental.pallas import tpu as pltpu
from jax.experimental.pallas import tpu_sc as plsc

N = 10000
E = 320000
D = 128
G = 64
BN_EPS = 1e-5

NC = 2
NS = 16
NW = NC * NS
EPT = E // NW
CH = 80
NSTEP = EPT // CH
RPS = N // NS

_MESH = plsc.VectorSubcoreMesh(core_axis_name="c", subcore_axis_name="s")


@functools.partial(
    pl.kernel,
    out_type=jax.ShapeDtypeStruct((NC, N, D), jnp.float32),
    mesh=_MESH,
    scratch_types=[
        pltpu.VMEM((EPT,), jnp.int32),
        pltpu.VMEM((NSTEP, CH), jnp.int32),
        pltpu.VMEM((CH, D), jnp.float32),
        pltpu.VMEM_SHARED((N, D), jnp.float32),
        pltpu.SemaphoreType.DMA,
    ],
)
def _sc_aggr_kernel(x_hbm, src_hbm, dst_hbm, z_hbm, out_hbm,
                    sidx, didx, rows, accum, sem):
    c = lax.axis_index("c")
    s = lax.axis_index("s")
    wid = s * NC + c
    pltpu.sync_copy(src_hbm.at[wid], sidx)
    pltpu.sync_copy(dst_hbm.at[wid], didx)
    pltpu.sync_copy(z_hbm, accum.at[pl.ds(s * RPS, RPS)])
    plsc.subcore_barrier()

    @pl.loop(0, NSTEP)
    def _(i):
        pltpu.async_copy(x_hbm.at[sidx.at[pl.ds(i * CH, CH)]], rows, sem).wait()
        pltpu.sync_copy(rows, accum.at[didx.at[i]], add=True)

    plsc.subcore_barrier()
    pltpu.sync_copy(accum.at[pl.ds(s * RPS, RPS)],
                    out_hbm.at[c, pl.ds(s * RPS, RPS)])


def _mlp_body(x_ref, a0_ref, a1_ref, w1_ref, b1_ref, w2_ref, b2_ref, o_ref):
    h = x_ref[...] + a0_ref[...] + a1_ref[...]
    h = jnp.dot(h, w1_ref[...], preferred_element_type=jnp.float32) + b1_ref[...]
    h = jnp.maximum(h, 0.0)
    h = jnp.dot(h, w2_ref[...], preferred_element_type=jnp.float32) + b2_ref[...]
    o_ref[...] = jnp.maximum(h, 0.0)


_BR = 1000


def _mlp(x, a0, a1, w1, b1, w2, b2):
    row = pl.BlockSpec((_BR, D), lambda i: (i, 0))
    mat = pl.BlockSpec((D, D), lambda i: (0, 0))
    vec = pl.BlockSpec((1, D), lambda i: (0, 0))
    return pl.pallas_call(
        _mlp_body,
        grid=(N // _BR,),
        in_specs=[row, row, row, mat, vec, mat, vec],
        out_specs=row,
        out_shape=jax.ShapeDtypeStruct((N, D), jnp.float32),
    )(x, a0, a1, w1, b1.reshape(1, D), w2, b2.reshape(1, D))


def _pool_body(h_ref, b_ref, w_ref, bb_ref, o_ref):
    gids = lax.broadcasted_iota(jnp.int32, (G, N), 0)
    onehot = (gids == b_ref[...]).astype(jnp.float32)
    pooled = jnp.dot(onehot, h_ref[...], preferred_element_type=jnp.float32)
    o_ref[...] = jnp.dot(pooled, w_ref[...],
                         preferred_element_type=jnp.float32) + bb_ref[...]


def _pool(h, batch2d, lin_w, lin_b):
    return pl.pallas_call(
        _pool_body,
        out_shape=jax.ShapeDtypeStruct((G, 1), jnp.float32),
    )(h, batch2d, lin_w, lin_b.reshape(1, 1))


def kernel(x, edge_index, batch,
           c1_W1, c1_b1, c1_g, c1_be, c1_W2, c1_b2,
           c2_W1, c2_b1, c2_g, c2_be, c2_W2, c2_b2,
           c3_W1, c3_b1, c3_g, c3_be, c3_W2, c3_b2,
           lin_W, lin_b):
    src = edge_index[0].reshape(NW, EPT)
    dst = edge_index[1].reshape(NW, NSTEP, CH)
    zrows = jnp.zeros((RPS, D), jnp.float32)

    h = x
    for w1, b1, g, be, w2, b2 in (
        (c1_W1, c1_b1, c1_g, c1_be, c1_W2, c1_b2),
        (c2_W1, c2_b1, c2_g, c2_be, c2_W2, c2_b2),
        (c3_W1, c3_b1, c3_g, c3_be, c3_W2, c3_b2),
    ):
        sc = g * (1.0 / jnp.sqrt(jnp.float32(1.0 + BN_EPS)))
        w1f = w1 * sc[None, :]
        b1f = b1 * sc + be
        agg = _sc_aggr_kernel(h, src, dst, zrows)
        h = _mlp(h, agg[0], agg[1], w1f, b1f, w2, b2)

    return _pool(h, batch.reshape(1, N), lin_W, lin_b)


# speedup vs baseline: 7.0085x; 7.0085x over previous
import functools

import jax
import jax.numpy as jnp
from jax import lax
from jax.experimental import pallas as pl
from jax.experimental.pallas import tpu as pltpu
from jax.experimental.pallas import tpu_sc as plsc

N = 10000
E = 320000
D = 128
G = 64
BN_EPS = 1e-5

NC = 2
NS = 16
NW = NC * NS
EPT = E // NW
CH = 80
NSTEP = EPT // CH
RPS = 624
TAIL = N - NS * RPS

_MESH = plsc.VectorSubcoreMesh(core_axis_name="c", subcore_axis_name="s")


@functools.partial(
    pl.kernel,
    out_type=jax.ShapeDtypeStruct((NC, N, D), jnp.float32),
    mesh=_MESH,
    scratch_types=[
        pltpu.VMEM((EPT,), jnp.int32),
        pltpu.VMEM((NSTEP, CH), jnp.int32),
        pltpu.VMEM((CH, D), jnp.float32),
        pltpu.VMEM_SHARED((N, D), jnp.float32),
        pltpu.SemaphoreType.DMA,
    ],
)
def _sc_aggr_kernel(x_hbm, src_hbm, dst_hbm, z_hbm, out_hbm,
                    sidx, didx, rows, accum, sem):
    c = lax.axis_index("c")
    s = lax.axis_index("s")
    wid = s * NC + c
    pltpu.sync_copy(src_hbm.at[wid], sidx)
    pltpu.sync_copy(dst_hbm.at[wid], didx)
    pltpu.sync_copy(z_hbm, accum.at[pl.ds(s * RPS, RPS)])

    @pl.when(s == NS - 1)
    def _():
        pltpu.sync_copy(z_hbm.at[pl.ds(0, TAIL)],
                        accum.at[pl.ds(NS * RPS, TAIL)])

    plsc.subcore_barrier()

    @pl.loop(0, NSTEP)
    def _(i):
        pltpu.async_copy(x_hbm.at[sidx.at[pl.ds(i * CH, CH)]], rows, sem).wait()
        pltpu.sync_copy(rows, accum.at[didx.at[i]], add=True)

    plsc.subcore_barrier()
    pltpu.sync_copy(accum.at[pl.ds(s * RPS, RPS)],
                    out_hbm.at[c, pl.ds(s * RPS, RPS)])

    @pl.when(s == NS - 1)
    def _():
        pltpu.sync_copy(accum.at[pl.ds(NS * RPS, TAIL)],
                        out_hbm.at[c, pl.ds(NS * RPS, TAIL)])


def _mlp_body(x_ref, a0_ref, a1_ref, w1_ref, b1_ref, w2_ref, b2_ref, o_ref):
    h = x_ref[...] + a0_ref[...] + a1_ref[...]
    h = jnp.dot(h, w1_ref[...], preferred_element_type=jnp.float32) + b1_ref[...]
    h = jnp.maximum(h, 0.0)
    h = jnp.dot(h, w2_ref[...], preferred_element_type=jnp.float32) + b2_ref[...]
    o_ref[...] = jnp.maximum(h, 0.0)


_BR = 1000


def _mlp(x, a0, a1, w1, b1, w2, b2):
    row = pl.BlockSpec((_BR, D), lambda i: (i, 0))
    mat = pl.BlockSpec((D, D), lambda i: (0, 0))
    vec = pl.BlockSpec((1, D), lambda i: (0, 0))
    return pl.pallas_call(
        _mlp_body,
        grid=(N // _BR,),
        in_specs=[row, row, row, mat, vec, mat, vec],
        out_specs=row,
        out_shape=jax.ShapeDtypeStruct((N, D), jnp.float32),
    )(x, a0, a1, w1, b1.reshape(1, D), w2, b2.reshape(1, D))


def _pool_body(h_ref, b_ref, w_ref, bb_ref, o_ref):
    gids = lax.broadcasted_iota(jnp.int32, (G, N), 0)
    onehot = (gids == b_ref[...]).astype(jnp.float32)
    pooled = jnp.dot(onehot, h_ref[...], preferred_element_type=jnp.float32)
    o_ref[...] = jnp.dot(pooled, w_ref[...],
                         preferred_element_type=jnp.float32) + bb_ref[...]


def _pool(h, batch2d, lin_w, lin_b):
    return pl.pallas_call(
        _pool_body,
        out_shape=jax.ShapeDtypeStruct((G, 1), jnp.float32),
    )(h, batch2d, lin_w, lin_b.reshape(1, 1))


def kernel(x, edge_index, batch,
           c1_W1, c1_b1, c1_g, c1_be, c1_W2, c1_b2,
           c2_W1, c2_b1, c2_g, c2_be, c2_W2, c2_b2,
           c3_W1, c3_b1, c3_g, c3_be, c3_W2, c3_b2,
           lin_W, lin_b):
    src = edge_index[0].reshape(NW, EPT)
    dst = edge_index[1].reshape(NW, NSTEP, CH)
    zrows = jnp.zeros((RPS, D), jnp.float32)

    h = x
    for w1, b1, g, be, w2, b2 in (
        (c1_W1, c1_b1, c1_g, c1_be, c1_W2, c1_b2),
        (c2_W1, c2_b1, c2_g, c2_be, c2_W2, c2_b2),
        (c3_W1, c3_b1, c3_g, c3_be, c3_W2, c3_b2),
    ):
        sc = g * (1.0 / jnp.sqrt(jnp.float32(1.0 + BN_EPS)))
        w1f = w1 * sc[None, :]
        b1f = b1 * sc + be
        agg = _sc_aggr_kernel(h, src, dst, zrows)
        h = _mlp(h, agg[0], agg[1], w1f, b1f, w2, b2)

    return _pool(h, batch.reshape(1, N), lin_W, lin_b)


# speedup vs baseline: 11.5284x; 1.6449x over previous
import functools

import jax
import jax.numpy as jnp
from jax import lax
from jax.experimental import pallas as pl
from jax.experimental.pallas import tpu as pltpu
from jax.experimental.pallas import tpu_sc as plsc

N = 10000
E = 320000
D = 128
G = 64
BN_EPS = 1e-5

NC = 2
NS = 16
NW = NC * NS
EPT = E // NW
CH = 40
NSTEP = EPT // CH
RPS = 624
TAIL = N - NS * RPS

_MESH = plsc.VectorSubcoreMesh(core_axis_name="c", subcore_axis_name="s")


NBUF = 5


@functools.partial(
    pl.kernel,
    out_type=jax.ShapeDtypeStruct((NC, N, D), jnp.float32),
    mesh=_MESH,
    scratch_types=[
        pltpu.VMEM((EPT,), jnp.int32),
        pltpu.VMEM((NBUF, CH), jnp.int32),
        pltpu.VMEM((NBUF, CH, D), jnp.float32),
        pltpu.VMEM_SHARED((N, D), jnp.float32),
        pltpu.SemaphoreType.DMA((NBUF,)),
        pltpu.SemaphoreType.DMA((NBUF,)),
        pltpu.SemaphoreType.DMA((NBUF,)),
    ],
)
def _sc_aggr_kernel(x_hbm, src_hbm, dst_hbm, z_hbm, out_hbm,
                    sidx, didxr, rows, accum, gsem, ssem, isem):
    c = lax.axis_index("c")
    s = lax.axis_index("s")
    wid = s * NC + c
    ebase = wid * EPT
    pltpu.sync_copy(src_hbm.at[pl.ds(ebase, EPT)], sidx)

    def start_gather(step, b):
        pltpu.async_copy(x_hbm.at[sidx.at[pl.ds(step * CH, CH)]],
                         rows.at[b], gsem.at[b])
        pltpu.async_copy(dst_hbm.at[pl.ds(ebase + step * CH, CH)],
                         didxr.at[b], isem.at[b])

    def wait_gather(b):
        pltpu.make_async_copy(x_hbm.at[pl.ds(0, CH)], rows.at[b],
                              gsem.at[b]).wait()
        pltpu.make_async_copy(dst_hbm.at[pl.ds(0, CH)], didxr.at[b],
                              isem.at[b]).wait()

    def wait_scatter(b):
        pltpu.make_async_copy(rows.at[b], accum.at[pl.ds(0, CH)],
                              ssem.at[b]).wait()

    for b in range(NBUF):
        start_gather(b, b)

    pltpu.sync_copy(z_hbm, accum.at[pl.ds(s * RPS, RPS)])

    @pl.when(s == NS - 1)
    def _():
        pltpu.sync_copy(z_hbm.at[pl.ds(0, TAIL)],
                        accum.at[pl.ds(NS * RPS, TAIL)])

    plsc.subcore_barrier()

    @pl.loop(0, NSTEP, step=NBUF)
    def _(i0):
        for b in range(NBUF):
            wait_gather(b)
            pltpu.async_copy(rows.at[b], accum.at[didxr.at[b]],
                             ssem.at[b], add=True)
        for b in range(NBUF):
            @pl.when(i0 + b + NBUF < NSTEP)
            def _():
                wait_scatter(b)
                start_gather(i0 + b + NBUF, b)

    for b in range(NBUF):
        wait_scatter(b)

    plsc.subcore_barrier()
    pltpu.sync_copy(accum.at[pl.ds(s * RPS, RPS)],
                    out_hbm.at[c, pl.ds(s * RPS, RPS)])

    @pl.when(s == NS - 1)
    def _():
        pltpu.sync_copy(accum.at[pl.ds(NS * RPS, TAIL)],
                        out_hbm.at[c, pl.ds(NS * RPS, TAIL)])


def _mlp_body(x_ref, a0_ref, a1_ref, w1_ref, b1_ref, w2_ref, b2_ref, o_ref):
    h = x_ref[...] + a0_ref[...] + a1_ref[...]
    h = jnp.dot(h, w1_ref[...], preferred_element_type=jnp.float32) + b1_ref[...]
    h = jnp.maximum(h, 0.0)
    h = jnp.dot(h, w2_ref[...], preferred_element_type=jnp.float32) + b2_ref[...]
    o_ref[...] = jnp.maximum(h, 0.0)


_BR = 1000


def _mlp(x, a0, a1, w1, b1, w2, b2):
    row = pl.BlockSpec((_BR, D), lambda i: (i, 0))
    mat = pl.BlockSpec((D, D), lambda i: (0, 0))
    vec = pl.BlockSpec((1, D), lambda i: (0, 0))
    return pl.pallas_call(
        _mlp_body,
        grid=(N // _BR,),
        in_specs=[row, row, row, mat, vec, mat, vec],
        out_specs=row,
        out_shape=jax.ShapeDtypeStruct((N, D), jnp.float32),
    )(x, a0, a1, w1, b1.reshape(1, D), w2, b2.reshape(1, D))


def _pool_body(h_ref, b_ref, w_ref, bb_ref, o_ref):
    gids = lax.broadcasted_iota(jnp.int32, (G, N), 0)
    onehot = (gids == b_ref[...]).astype(jnp.float32)
    pooled = jnp.dot(onehot, h_ref[...], preferred_element_type=jnp.float32)
    o_ref[...] = jnp.dot(pooled, w_ref[...],
                         preferred_element_type=jnp.float32) + bb_ref[...]


def _pool(h, batch2d, lin_w, lin_b):
    return pl.pallas_call(
        _pool_body,
        out_shape=jax.ShapeDtypeStruct((G, 1), jnp.float32),
    )(h, batch2d, lin_w, lin_b.reshape(1, 1))


def kernel(x, edge_index, batch,
           c1_W1, c1_b1, c1_g, c1_be, c1_W2, c1_b2,
           c2_W1, c2_b1, c2_g, c2_be, c2_W2, c2_b2,
           c3_W1, c3_b1, c3_g, c3_be, c3_W2, c3_b2,
           lin_W, lin_b):
    src = edge_index[0]
    dst = edge_index[1]
    zrows = jnp.zeros((RPS, D), jnp.float32)

    h = x
    for w1, b1, g, be, w2, b2 in (
        (c1_W1, c1_b1, c1_g, c1_be, c1_W2, c1_b2),
        (c2_W1, c2_b1, c2_g, c2_be, c2_W2, c2_b2),
        (c3_W1, c3_b1, c3_g, c3_be, c3_W2, c3_b2),
    ):
        sc = g * (1.0 / jnp.sqrt(jnp.float32(1.0 + BN_EPS)))
        w1f = w1 * sc[None, :]
        b1f = b1 * sc + be
        agg = _sc_aggr_kernel(h, src, dst, zrows)
        h = _mlp(h, agg[0], agg[1], w1f, b1f, w2, b2)

    return _pool(h, batch.reshape(1, N), lin_W, lin_b)


# speedup vs baseline: 12.3044x; 1.0673x over previous
import functools

import jax
import jax.numpy as jnp
from jax import lax
from jax.experimental import pallas as pl
from jax.experimental.pallas import tpu as pltpu
from jax.experimental.pallas import tpu_sc as plsc

N = 10000
E = 320000
D = 128
G = 64
BN_EPS = 1e-5

NC = 2
NS = 16
NW = NC * NS
EPT = E // NW
CH = 40
NSTEP = EPT // CH
RPS = 624
TAIL = N - NS * RPS

_MESH = plsc.VectorSubcoreMesh(core_axis_name="c", subcore_axis_name="s")


NBUF = 5


@functools.partial(
    pl.kernel,
    out_type=jax.ShapeDtypeStruct((NC, N, D), jnp.float32),
    mesh=_MESH,
    scratch_types=[
        pltpu.VMEM((EPT,), jnp.int32),
        pltpu.VMEM((NBUF, CH), jnp.int32),
        pltpu.VMEM((NBUF, CH, D), jnp.float32),
        pltpu.VMEM_SHARED((N, D), jnp.float32),
        pltpu.SemaphoreType.DMA((NBUF,)),
        pltpu.SemaphoreType.DMA((NBUF,)),
        pltpu.SemaphoreType.DMA((NBUF,)),
    ],
)
def _sc_aggr_kernel(x_hbm, src_hbm, dst_hbm, z_hbm, out_hbm,
                    sidx, didxr, rows, accum, gsem, ssem, isem):
    c = lax.axis_index("c")
    s = lax.axis_index("s")
    wid = s * NC + c
    ebase = wid * EPT
    pltpu.sync_copy(src_hbm.at[pl.ds(ebase, EPT)], sidx)

    def start_gather(step, b):
        pltpu.async_copy(x_hbm.at[sidx.at[pl.ds(step * CH, CH)]],
                         rows.at[b], gsem.at[b])
        pltpu.async_copy(dst_hbm.at[pl.ds(ebase + step * CH, CH)],
                         didxr.at[b], isem.at[b])

    def wait_gather(b):
        pltpu.make_async_copy(x_hbm.at[pl.ds(0, CH)], rows.at[b],
                              gsem.at[b]).wait()
        pltpu.make_async_copy(dst_hbm.at[pl.ds(0, CH)], didxr.at[b],
                              isem.at[b]).wait()

    def wait_scatter(b):
        pltpu.make_async_copy(rows.at[b], accum.at[pl.ds(0, CH)],
                              ssem.at[b]).wait()

    for b in range(NBUF):
        start_gather(b, b)

    pltpu.sync_copy(z_hbm, accum.at[pl.ds(s * RPS, RPS)])

    @pl.when(s == NS - 1)
    def _():
        pltpu.sync_copy(z_hbm.at[pl.ds(0, TAIL)],
                        accum.at[pl.ds(NS * RPS, TAIL)])

    plsc.subcore_barrier()

    @pl.loop(0, NSTEP, step=NBUF)
    def _(i0):
        for b in range(NBUF):
            wait_gather(b)
            pltpu.async_copy(rows.at[b], accum.at[didxr.at[b]],
                             ssem.at[b], add=True)
        for b in range(NBUF):
            @pl.when(i0 + b + NBUF < NSTEP)
            def _():
                wait_scatter(b)
                start_gather(i0 + b + NBUF, b)

    for b in range(NBUF):
        wait_scatter(b)

    plsc.subcore_barrier()
    pltpu.sync_copy(accum.at[pl.ds(s * RPS, RPS)],
                    out_hbm.at[c, pl.ds(s * RPS, RPS)])

    @pl.when(s == NS - 1)
    def _():
        pltpu.sync_copy(accum.at[pl.ds(NS * RPS, TAIL)],
                        out_hbm.at[c, pl.ds(NS * RPS, TAIL)])


_BR = 1000


def _mlp_core(x_ref, a_ref, w1_ref, b1_ref, w2_ref, b2_ref):
    h = x_ref[...] + a_ref[0] + a_ref[1]
    h = jnp.dot(h, w1_ref[...], preferred_element_type=jnp.float32) + b1_ref[...]
    h = jnp.maximum(h, 0.0)
    h = jnp.dot(h, w2_ref[...], preferred_element_type=jnp.float32) + b2_ref[...]
    return jnp.maximum(h, 0.0)


def _mlp_body(x_ref, a_ref, w1_ref, b1_ref, w2_ref, b2_ref, o_ref):
    o_ref[...] = _mlp_core(x_ref, a_ref, w1_ref, b1_ref, w2_ref, b2_ref)


_ROW = pl.BlockSpec((_BR, D), lambda i: (i, 0))
_AGG = pl.BlockSpec((NC, _BR, D), lambda i: (0, i, 0))
_MAT = pl.BlockSpec((D, D), lambda i: (0, 0))
_VEC = pl.BlockSpec((1, D), lambda i: (0, 0))


def _mlp(x, agg, w1, b1, w2, b2):
    return pl.pallas_call(
        _mlp_body,
        grid=(N // _BR,),
        in_specs=[_ROW, _AGG, _MAT, _VEC, _MAT, _VEC],
        out_specs=_ROW,
        out_shape=jax.ShapeDtypeStruct((N, D), jnp.float32),
    )(x, agg, w1, b1.reshape(1, D), w2, b2.reshape(1, D))


def _mlp_pool_body(x_ref, a_ref, w1_ref, b1_ref, w2_ref, b2_ref,
                   batch_ref, w_ref, bb_ref, o_ref, acc_ref):
    i = pl.program_id(0)
    h = _mlp_core(x_ref, a_ref, w1_ref, b1_ref, w2_ref, b2_ref)
    seg = batch_ref[...].reshape(1, _BR)
    gids = lax.broadcasted_iota(jnp.int32, (G, _BR), 0)
    onehot = (gids == seg).astype(jnp.float32)
    part = jnp.dot(onehot, h, preferred_element_type=jnp.float32)

    @pl.when(i == 0)
    def _():
        acc_ref[...] = jnp.zeros_like(acc_ref)

    acc_ref[...] += part

    @pl.when(i == pl.num_programs(0) - 1)
    def _():
        o_ref[...] = jnp.dot(acc_ref[...], w_ref[...],
                             preferred_element_type=jnp.float32) + bb_ref[...]


def _mlp_pool(x, agg, w1, b1, w2, b2, batch3d, lin_w, lin_b):
    return pl.pallas_call(
        _mlp_pool_body,
        grid=(N // _BR,),
        in_specs=[_ROW, _AGG, _MAT, _VEC, _MAT, _VEC,
                  pl.BlockSpec((1, 1, _BR), lambda i: (i, 0, 0)),
                  pl.BlockSpec((D, 1), lambda i: (0, 0)),
                  pl.BlockSpec((1, 1), lambda i: (0, 0))],
        out_specs=pl.BlockSpec((G, 1), lambda i: (0, 0)),
        out_shape=jax.ShapeDtypeStruct((G, 1), jnp.float32),
        scratch_shapes=[pltpu.VMEM((G, D), jnp.float32)],
    )(x, agg, w1, b1.reshape(1, D), w2, b2.reshape(1, D),
      batch3d, lin_w, lin_b.reshape(1, 1))


def kernel(x, edge_index, batch,
           c1_W1, c1_b1, c1_g, c1_be, c1_W2, c1_b2,
           c2_W1, c2_b1, c2_g, c2_be, c2_W2, c2_b2,
           c3_W1, c3_b1, c3_g, c3_be, c3_W2, c3_b2,
           lin_W, lin_b):
    src = edge_index[0]
    dst = edge_index[1]
    zrows = jnp.zeros((RPS, D), jnp.float32)

    def fold_bn(w1, b1, g, be):
        s = g * (1.0 / jnp.sqrt(jnp.float32(1.0 + BN_EPS)))
        return w1 * s[None, :], b1 * s + be

    h = x
    for w1, b1, g, be, w2, b2 in (
        (c1_W1, c1_b1, c1_g, c1_be, c1_W2, c1_b2),
        (c2_W1, c2_b1, c2_g, c2_be, c2_W2, c2_b2),
    ):
        w1f, b1f = fold_bn(w1, b1, g, be)
        agg = _sc_aggr_kernel(h, src, dst, zrows)
        h = _mlp(h, agg, w1f, b1f, w2, b2)

    w1f, b1f = fold_bn(c3_W1, c3_b1, c3_g, c3_be)
    agg = _sc_aggr_kernel(h, src, dst, zrows)
    return _mlp_pool(h, agg, w1f, b1f, c3_W2, c3_b2,
                     batch.reshape(N // _BR, 1, _BR), lin_W, lin_b)
